# initial kernel scaffold (unmeasured)
import jax
import jax.numpy as jnp
from jax import lax
from jax.experimental import pallas as pl
from jax.experimental.pallas import tpu as pltpu

N = 32

_sem_signal = getattr(pl, "semaphore_signal", None) or pltpu.semaphore_signal
_sem_wait = getattr(pl, "semaphore_wait", None) or pltpu.semaphore_wait
_DevId = getattr(pl, "DeviceIdType", None) or pltpu.DeviceIdType


def kernel(x, w_mat, scale_x, scale_w):
    M, _ = x.shape
    Nout = w_mat.shape[1]
    CH = M // N

    partial = lax.dot_general(
        x, w_mat, dimension_numbers=(((1,), (0,)), ((), ())),
        preferred_element_type=jnp.int32)
    s = (scale_x * scale_w).reshape(1, 1)

    def body(partial_ref, s_ref, out_ref, sendbuf, recvbuf, pbuf, fbuf,
             rs_ssem, rs_rsem, ag_ssem, ag_rsem, credit, ldma):
        me = lax.axis_index("i")
        left = lax.rem(me + (N - 1), N)
        right = lax.rem(me + 1, N)

        cp = pltpu.make_async_copy(
            partial_ref.at[pl.ds(me * CH, CH), :], sendbuf, ldma)
        cp.start()
        cp.wait()

        for h in range(N - 1):
            slot = h % 2
            if h >= 2:
                _sem_wait(credit.at[slot], 1)
            rdma = pltpu.make_async_remote_copy(
                src_ref=sendbuf,
                dst_ref=recvbuf.at[slot],
                send_sem=rs_ssem.at[h],
                recv_sem=rs_rsem.at[h],
                device_id=(right,),
                device_id_type=_DevId.MESH)
            rdma.start()
            rc = lax.rem(me + (N - 1 - h), N)
            cp = pltpu.make_async_copy(
                partial_ref.at[pl.ds(rc * CH, CH), :], pbuf, ldma)
            cp.start()
            cp.wait()
            rdma.wait()
            sendbuf[...] = recvbuf[slot] + pbuf[...]
            _sem_signal(credit.at[slot], inc=1, device_id=(left,),
                        device_id_type=_DevId.MESH)
        _sem_wait(credit.at[(N - 1) % 2], 1)
        _sem_wait(credit.at[N % 2], 1)

        y = sendbuf[...].astype(jnp.float32) * s_ref[0, 0]
        fbuf[...] = y / (1.0 + jnp.exp(-jnp.clip(y, -60.0, 60.0)))
        cp = pltpu.make_async_copy(
            fbuf, out_ref.at[pl.ds(right * CH, CH), :], ldma)
        cp.start()
        cp.wait()

        for h in range(N - 1):
            cs = lax.rem(me + (N + 1 - h), N)
            sl = out_ref.at[pl.ds(cs * CH, CH), :]
            rdma = pltpu.make_async_remote_copy(
                src_ref=sl, dst_ref=sl,
                send_sem=ag_ssem.at[h], recv_sem=ag_rsem.at[h],
                device_id=(right,), device_id_type=_DevId.MESH)
            rdma.start()
            rdma.wait()

    return pl.pallas_call(
        body,
        out_shape=jax.ShapeDtypeStruct((M, Nout), jnp.float32),
        in_specs=[pl.BlockSpec(memory_space=pltpu.ANY),
                  pl.BlockSpec(memory_space=pltpu.SMEM)],
        out_specs=pl.BlockSpec(memory_space=pltpu.ANY),
        scratch_shapes=[
            pltpu.VMEM((CH, Nout), jnp.int32),
            pltpu.VMEM((2, CH, Nout), jnp.int32),
            pltpu.VMEM((CH, Nout), jnp.int32),
            pltpu.VMEM((CH, Nout), jnp.float32),
            pltpu.SemaphoreType.DMA((N - 1,)),
            pltpu.SemaphoreType.DMA((N - 1,)),
            pltpu.SemaphoreType.DMA((N - 1,)),
            pltpu.SemaphoreType.DMA((N - 1,)),
            pltpu.SemaphoreType.REGULAR((2,)),
            pltpu.SemaphoreType.DMA,
        ],
        compiler_params=pltpu.CompilerParams(collective_id=0),
    )(partial, s)


# baseline (device time: 3077896 ns/iter reference)
import jax
import jax.numpy as jnp
from jax import lax
from jax.experimental import pallas as pl
from jax.experimental.pallas import tpu as pltpu

N = 32

_sem_signal = getattr(pl, "semaphore_signal", None) or pltpu.semaphore_signal
_sem_wait = getattr(pl, "semaphore_wait", None) or pltpu.semaphore_wait
_DevId = getattr(pl, "DeviceIdType", None) or pltpu.DeviceIdType


def kernel(x, w_mat, scale_x, scale_w):
    M, _ = x.shape
    Nout = w_mat.shape[1]
    CH = M // N

    partial = lax.dot_general(
        x, w_mat, dimension_numbers=(((1,), (0,)), ((), ())),
        preferred_element_type=jnp.int32)
    s = (scale_x * scale_w).reshape(1, 1)

    def body(partial_ref, s_ref, out_ref, sendbuf, recvbuf, pbuf, fbuf,
             rs_ssem, rs_rsem, ag_ssem, ag_rsem, credit, ldma):
        me = lax.axis_index("i")
        left = lax.rem(me + (N - 1), N)
        right = lax.rem(me + 1, N)

        cp = pltpu.make_async_copy(
            partial_ref.at[pl.ds(me * CH, CH), :], sendbuf, ldma)
        cp.start()
        cp.wait()

        for h in range(N - 1):
            slot = h % 2
            if h >= 2:
                _sem_wait(credit.at[slot], 1)
            rdma = pltpu.make_async_remote_copy(
                src_ref=sendbuf,
                dst_ref=recvbuf.at[slot],
                send_sem=rs_ssem.at[h],
                recv_sem=rs_rsem.at[h],
                device_id=(right,),
                device_id_type=_DevId.MESH)
            rdma.start()
            rc = lax.rem(me + (N - 1 - h), N)
            cp = pltpu.make_async_copy(
                partial_ref.at[pl.ds(rc * CH, CH), :], pbuf, ldma)
            cp.start()
            cp.wait()
            rdma.wait()
            sendbuf[...] = recvbuf[slot] + pbuf[...]
            _sem_signal(credit.at[slot], inc=1, device_id=(left,),
                        device_id_type=_DevId.MESH)
        _sem_wait(credit.at[(N - 1) % 2], 1)
        _sem_wait(credit.at[N % 2], 1)

        y = sendbuf[...].astype(jnp.float32) * s_ref[0, 0]
        fbuf[...] = y / (1.0 + jnp.exp(-jnp.clip(y, -60.0, 60.0)))
        cp = pltpu.make_async_copy(
            fbuf, out_ref.at[pl.ds(right * CH, CH), :], ldma)
        cp.start()
        cp.wait()

        for h in range(N - 1):
            cs = lax.rem(me + (N + 1 - h), N)
            sl = out_ref.at[pl.ds(cs * CH, CH), :]
            rdma = pltpu.make_async_remote_copy(
                src_ref=sl, dst_ref=sl,
                send_sem=ag_ssem.at[h], recv_sem=ag_rsem.at[h],
                device_id=(right,), device_id_type=_DevId.MESH)
            rdma.start()
            rdma.wait()

    return pl.pallas_call(
        body,
        out_shape=jax.ShapeDtypeStruct((M, Nout), jnp.float32),
        in_specs=[pl.BlockSpec(memory_space=pl.ANY),
                  pl.BlockSpec(memory_space=pltpu.SMEM)],
        out_specs=pl.BlockSpec(memory_space=pl.ANY),
        scratch_shapes=[
            pltpu.VMEM((CH, Nout), jnp.int32),
            pltpu.VMEM((2, CH, Nout), jnp.int32),
            pltpu.VMEM((CH, Nout), jnp.int32),
            pltpu.VMEM((CH, Nout), jnp.float32),
            pltpu.SemaphoreType.DMA((N - 1,)),
            pltpu.SemaphoreType.DMA((N - 1,)),
            pltpu.SemaphoreType.DMA((N - 1,)),
            pltpu.SemaphoreType.DMA((N - 1,)),
            pltpu.SemaphoreType.REGULAR((2,)),
            pltpu.SemaphoreType.DMA,
        ],
    )(partial, s)


# device time: 3065738 ns/iter; 1.0040x vs baseline; 1.0040x over previous
import jax
import jax.numpy as jnp
from jax import lax
from jax.experimental import pallas as pl
from jax.experimental.pallas import tpu as pltpu

N = 32

_sem_signal = getattr(pl, "semaphore_signal", None) or pltpu.semaphore_signal
_sem_wait = getattr(pl, "semaphore_wait", None) or pltpu.semaphore_wait
_DevId = getattr(pl, "DeviceIdType", None) or pltpu.DeviceIdType


def kernel(x, w_mat, scale_x, scale_w):
    M, _ = x.shape
    Nout = w_mat.shape[1]
    CH = M // N
    H = Nout // 2

    partial = lax.dot_general(
        x, w_mat, dimension_numbers=(((1,), (0,)), ((), ())),
        preferred_element_type=jnp.int32)
    s = (scale_x * scale_w).reshape(1, 1)

    def body(partial_ref, s_ref, out_ref,
             sendR, recvR, pR, fR, sendL, recvL, pL, fL,
             rs_ssR, rs_rsR, rs_ssL, rs_rsL,
             ag_ssR, ag_rsR, ag_ssL, ag_rsL,
             creditR, creditL, ldmaR, ldmaL):
        me = lax.axis_index("i")
        left = lax.rem(me + (N - 1), N)
        right = lax.rem(me + 1, N)

        cpR = pltpu.make_async_copy(
            partial_ref.at[pl.ds(me * CH, CH), pl.ds(0, H)], sendR, ldmaR)
        cpL = pltpu.make_async_copy(
            partial_ref.at[pl.ds(me * CH, CH), pl.ds(H, H)], sendL, ldmaL)
        cpR.start()
        cpL.start()
        cpR.wait()
        cpL.wait()

        for h in range(N - 1):
            slot = h % 2
            if h >= 2:
                _sem_wait(creditR.at[slot], 1)
                _sem_wait(creditL.at[slot], 1)
            rdmaR = pltpu.make_async_remote_copy(
                src_ref=sendR, dst_ref=recvR.at[slot],
                send_sem=rs_ssR.at[h], recv_sem=rs_rsR.at[h],
                device_id=(right,), device_id_type=_DevId.MESH)
            rdmaL = pltpu.make_async_remote_copy(
                src_ref=sendL, dst_ref=recvL.at[slot],
                send_sem=rs_ssL.at[h], recv_sem=rs_rsL.at[h],
                device_id=(left,), device_id_type=_DevId.MESH)
            rdmaR.start()
            rdmaL.start()
            rcR = lax.rem(me + (N - 1 - h), N)
            rcL = lax.rem(me + (h + 1), N)
            cpR = pltpu.make_async_copy(
                partial_ref.at[pl.ds(rcR * CH, CH), pl.ds(0, H)], pR, ldmaR)
            cpL = pltpu.make_async_copy(
                partial_ref.at[pl.ds(rcL * CH, CH), pl.ds(H, H)], pL, ldmaL)
            cpR.start()
            cpL.start()
            cpR.wait()
            cpL.wait()
            rdmaR.wait()
            rdmaL.wait()
            sendR[...] = recvR[slot] + pR[...]
            sendL[...] = recvL[slot] + pL[...]
            _sem_signal(creditR.at[slot], inc=1, device_id=(left,),
                        device_id_type=_DevId.MESH)
            _sem_signal(creditL.at[slot], inc=1, device_id=(right,),
                        device_id_type=_DevId.MESH)
        _sem_wait(creditR.at[(N - 1) % 2], 1)
        _sem_wait(creditR.at[N % 2], 1)
        _sem_wait(creditL.at[(N - 1) % 2], 1)
        _sem_wait(creditL.at[N % 2], 1)

        sc = s_ref[0, 0]
        yR = sendR[...].astype(jnp.float32) * sc
        fR[...] = yR / (1.0 + jnp.exp(-jnp.clip(yR, -60.0, 60.0)))
        yL = sendL[...].astype(jnp.float32) * sc
        fL[...] = yL / (1.0 + jnp.exp(-jnp.clip(yL, -60.0, 60.0)))
        cpR = pltpu.make_async_copy(
            fR, out_ref.at[pl.ds(right * CH, CH), pl.ds(0, H)], ldmaR)
        cpL = pltpu.make_async_copy(
            fL, out_ref.at[pl.ds(left * CH, CH), pl.ds(H, H)], ldmaL)
        cpR.start()
        cpL.start()
        cpR.wait()
        cpL.wait()

        for h in range(N - 1):
            csR = lax.rem(me + (N + 1 - h), N)
            csL = lax.rem(me + (N - 1 + h), N)
            slR = out_ref.at[pl.ds(csR * CH, CH), pl.ds(0, H)]
            slL = out_ref.at[pl.ds(csL * CH, CH), pl.ds(H, H)]
            rdmaR = pltpu.make_async_remote_copy(
                src_ref=slR, dst_ref=slR,
                send_sem=ag_ssR.at[h], recv_sem=ag_rsR.at[h],
                device_id=(right,), device_id_type=_DevId.MESH)
            rdmaL = pltpu.make_async_remote_copy(
                src_ref=slL, dst_ref=slL,
                send_sem=ag_ssL.at[h], recv_sem=ag_rsL.at[h],
                device_id=(left,), device_id_type=_DevId.MESH)
            rdmaR.start()
            rdmaL.start()
            rdmaR.wait()
            rdmaL.wait()

    return pl.pallas_call(
        body,
        out_shape=jax.ShapeDtypeStruct((M, Nout), jnp.float32),
        in_specs=[pl.BlockSpec(memory_space=pl.ANY),
                  pl.BlockSpec(memory_space=pltpu.SMEM)],
        out_specs=pl.BlockSpec(memory_space=pl.ANY),
        scratch_shapes=[
            pltpu.VMEM((CH, H), jnp.int32),
            pltpu.VMEM((2, CH, H), jnp.int32),
            pltpu.VMEM((CH, H), jnp.int32),
            pltpu.VMEM((CH, H), jnp.float32),
            pltpu.VMEM((CH, H), jnp.int32),
            pltpu.VMEM((2, CH, H), jnp.int32),
            pltpu.VMEM((CH, H), jnp.int32),
            pltpu.VMEM((CH, H), jnp.float32),
            pltpu.SemaphoreType.DMA((N - 1,)),
            pltpu.SemaphoreType.DMA((N - 1,)),
            pltpu.SemaphoreType.DMA((N - 1,)),
            pltpu.SemaphoreType.DMA((N - 1,)),
            pltpu.SemaphoreType.DMA((N - 1,)),
            pltpu.SemaphoreType.DMA((N - 1,)),
            pltpu.SemaphoreType.DMA((N - 1,)),
            pltpu.SemaphoreType.DMA((N - 1,)),
            pltpu.SemaphoreType.REGULAR((2,)),
            pltpu.SemaphoreType.REGULAR((2,)),
            pltpu.SemaphoreType.DMA,
            pltpu.SemaphoreType.DMA,
        ],
    )(partial, s)


# device time: 1683374 ns/iter; 1.8284x vs baseline; 1.8212x over previous
import jax
import jax.numpy as jnp
from jax import lax
from jax.experimental import pallas as pl
from jax.experimental.pallas import tpu as pltpu

N = 32

_sem_signal = getattr(pl, "semaphore_signal", None) or pltpu.semaphore_signal
_sem_wait = getattr(pl, "semaphore_wait", None) or pltpu.semaphore_wait
_DevId = getattr(pl, "DeviceIdType", None) or pltpu.DeviceIdType


def _ring_permutation():
    p_yz = []
    for z in range(4):
        ys = range(4) if z % 2 == 0 else range(3, -1, -1)
        p_yz.extend((y, z) for y in ys)
    cycle = [(0, y, z) for (y, z) in p_yz] + \
            [(1, y, z) for (y, z) in reversed(p_yz)]
    plane = [(0, 0), (1, 0), (1, 1), (0, 1), (0, 2), (1, 2), (1, 3), (0, 3)]
    return [8 * z + plane.index((x, y)) for (x, y, z) in cycle]


_PI = _ring_permutation()
_INV = [0] * N
for _p, _m in enumerate(_PI):
    _INV[_m] = _p


def kernel(x, w_mat, scale_x, scale_w):
    M, _ = x.shape
    Nout = w_mat.shape[1]
    CH = M // N
    H = Nout // 2

    partial = lax.dot_general(
        x, w_mat, dimension_numbers=(((1,), (0,)), ((), ())),
        preferred_element_type=jnp.int32)
    s = (scale_x * scale_w).reshape(1, 1)

    me = lax.axis_index("i")
    p = jnp.asarray(_INV, jnp.int32)[me]
    pi = jnp.asarray(_PI, jnp.int32)
    right_m = pi[lax.rem(p + 1, N)]
    left_m = pi[lax.rem(p + (N - 1), N)]
    meta = jnp.stack([p.astype(jnp.int32), left_m, right_m])

    def body(meta_ref, partial_ref, s_ref, out_ref,
             sendR, recvR, pR, fR, sendL, recvL, pL, fL,
             rs_ssR, rs_rsR, rs_ssL, rs_rsL,
             ag_ssR, ag_rsR, ag_ssL, ag_rsL,
             creditR, creditL, ldmaR, ldmaL):
        p = meta_ref[0]
        left = meta_ref[1]
        right = meta_ref[2]

        cpR = pltpu.make_async_copy(
            partial_ref.at[pl.ds(p * CH, CH), pl.ds(0, H)], sendR, ldmaR)
        cpL = pltpu.make_async_copy(
            partial_ref.at[pl.ds(p * CH, CH), pl.ds(H, H)], sendL, ldmaL)
        cpR.start()
        cpL.start()
        cpR.wait()
        cpL.wait()

        for h in range(N - 1):
            slot = h % 2
            if h >= 2:
                _sem_wait(creditR.at[slot], 1)
                _sem_wait(creditL.at[slot], 1)
            rdmaR = pltpu.make_async_remote_copy(
                src_ref=sendR, dst_ref=recvR.at[slot],
                send_sem=rs_ssR.at[h], recv_sem=rs_rsR.at[h],
                device_id=(right,), device_id_type=_DevId.MESH)
            rdmaL = pltpu.make_async_remote_copy(
                src_ref=sendL, dst_ref=recvL.at[slot],
                send_sem=rs_ssL.at[h], recv_sem=rs_rsL.at[h],
                device_id=(left,), device_id_type=_DevId.MESH)
            rdmaR.start()
            rdmaL.start()
            rcR = lax.rem(p + (N - 1 - h), N)
            rcL = lax.rem(p + (h + 1), N)
            cpR = pltpu.make_async_copy(
                partial_ref.at[pl.ds(rcR * CH, CH), pl.ds(0, H)], pR, ldmaR)
            cpL = pltpu.make_async_copy(
                partial_ref.at[pl.ds(rcL * CH, CH), pl.ds(H, H)], pL, ldmaL)
            cpR.start()
            cpL.start()
            cpR.wait()
            cpL.wait()
            rdmaR.wait()
            rdmaL.wait()
            sendR[...] = recvR[slot] + pR[...]
            sendL[...] = recvL[slot] + pL[...]
            _sem_signal(creditR.at[slot], inc=1, device_id=(left,),
                        device_id_type=_DevId.MESH)
            _sem_signal(creditL.at[slot], inc=1, device_id=(right,),
                        device_id_type=_DevId.MESH)
        _sem_wait(creditR.at[(N - 1) % 2], 1)
        _sem_wait(creditR.at[N % 2], 1)
        _sem_wait(creditL.at[(N - 1) % 2], 1)
        _sem_wait(creditL.at[N % 2], 1)

        ownR = lax.rem(p + 1, N)
        ownL = lax.rem(p + (N - 1), N)
        sc = s_ref[0, 0]
        yR = sendR[...].astype(jnp.float32) * sc
        fR[...] = yR / (1.0 + jnp.exp(-jnp.clip(yR, -60.0, 60.0)))
        yL = sendL[...].astype(jnp.float32) * sc
        fL[...] = yL / (1.0 + jnp.exp(-jnp.clip(yL, -60.0, 60.0)))
        cpR = pltpu.make_async_copy(
            fR, out_ref.at[pl.ds(ownR * CH, CH), pl.ds(0, H)], ldmaR)
        cpL = pltpu.make_async_copy(
            fL, out_ref.at[pl.ds(ownL * CH, CH), pl.ds(H, H)], ldmaL)
        cpR.start()
        cpL.start()
        cpR.wait()
        cpL.wait()

        for h in range(N - 1):
            csR = lax.rem(p + (N + 1 - h), N)
            csL = lax.rem(p + (N - 1 + h), N)
            slR = out_ref.at[pl.ds(csR * CH, CH), pl.ds(0, H)]
            slL = out_ref.at[pl.ds(csL * CH, CH), pl.ds(H, H)]
            rdmaR = pltpu.make_async_remote_copy(
                src_ref=slR, dst_ref=slR,
                send_sem=ag_ssR.at[h], recv_sem=ag_rsR.at[h],
                device_id=(right,), device_id_type=_DevId.MESH)
            rdmaL = pltpu.make_async_remote_copy(
                src_ref=slL, dst_ref=slL,
                send_sem=ag_ssL.at[h], recv_sem=ag_rsL.at[h],
                device_id=(left,), device_id_type=_DevId.MESH)
            rdmaR.start()
            rdmaL.start()
            rdmaR.wait()
            rdmaL.wait()

    return pl.pallas_call(
        body,
        out_shape=jax.ShapeDtypeStruct((M, Nout), jnp.float32),
        in_specs=[pl.BlockSpec(memory_space=pltpu.SMEM),
                  pl.BlockSpec(memory_space=pl.ANY),
                  pl.BlockSpec(memory_space=pltpu.SMEM)],
        out_specs=pl.BlockSpec(memory_space=pl.ANY),
        scratch_shapes=[
            pltpu.VMEM((CH, H), jnp.int32),
            pltpu.VMEM((2, CH, H), jnp.int32),
            pltpu.VMEM((CH, H), jnp.int32),
            pltpu.VMEM((CH, H), jnp.float32),
            pltpu.VMEM((CH, H), jnp.int32),
            pltpu.VMEM((2, CH, H), jnp.int32),
            pltpu.VMEM((CH, H), jnp.int32),
            pltpu.VMEM((CH, H), jnp.float32),
            pltpu.SemaphoreType.DMA((N - 1,)),
            pltpu.SemaphoreType.DMA((N - 1,)),
            pltpu.SemaphoreType.DMA((N - 1,)),
            pltpu.SemaphoreType.DMA((N - 1,)),
            pltpu.SemaphoreType.DMA((N - 1,)),
            pltpu.SemaphoreType.DMA((N - 1,)),
            pltpu.SemaphoreType.DMA((N - 1,)),
            pltpu.SemaphoreType.DMA((N - 1,)),
            pltpu.SemaphoreType.REGULAR((2,)),
            pltpu.SemaphoreType.REGULAR((2,)),
            pltpu.SemaphoreType.DMA,
            pltpu.SemaphoreType.DMA,
        ],
    )(meta, partial, s)


# device time: 1553732 ns/iter; 1.9810x vs baseline; 1.0834x over previous
import jax
import jax.numpy as jnp
from jax import lax
from jax.experimental import pallas as pl
from jax.experimental.pallas import tpu as pltpu

N = 32

_sem_signal = getattr(pl, "semaphore_signal", None) or pltpu.semaphore_signal
_sem_wait = getattr(pl, "semaphore_wait", None) or pltpu.semaphore_wait
_DevId = getattr(pl, "DeviceIdType", None) or pltpu.DeviceIdType


def _ring_permutation():
    p_yz = []
    for z in range(4):
        ys = range(4) if z % 2 == 0 else range(3, -1, -1)
        p_yz.extend((y, z) for y in ys)
    cycle = [(0, y, z) for (y, z) in p_yz] + \
            [(1, y, z) for (y, z) in reversed(p_yz)]
    plane = [(0, 0), (1, 0), (1, 1), (0, 1), (0, 2), (1, 2), (1, 3), (0, 3)]
    return [8 * z + plane.index((x, y)) for (x, y, z) in cycle]


_PI = _ring_permutation()
_INV = [0] * N
for _p, _m in enumerate(_PI):
    _INV[_m] = _p


def kernel(x, w_mat, scale_x, scale_w):
    M, _ = x.shape
    Nout = w_mat.shape[1]
    CH = M // N
    Q = Nout // 4

    partial = lax.dot_general(
        x, w_mat, dimension_numbers=(((1,), (0,)), ((), ())),
        preferred_element_type=jnp.int32)
    s = (scale_x * scale_w).reshape(1, 1)

    me = lax.axis_index("i")
    p = jnp.asarray(_INV, jnp.int32)[me]
    pi = jnp.asarray(_PI, jnp.int32)
    right_m = pi[lax.rem(p + 1, N)]
    left_m = pi[lax.rem(p + (N - 1), N)]
    meta = jnp.stack([p.astype(jnp.int32), left_m, right_m])

    def body(meta_ref, partial_ref, s_ref, out_ref,
             sendb, recvb, pbuf, fbuf,
             rs_ss, rs_rs, ag_ss, ag_rs, credit, ldma, pdma):
        p = meta_ref[0]
        left = meta_ref[1]
        right = meta_ref[2]

        PIPES = (0, 2, 1, 3)

        def is_r(i):
            return i < 2

        def dev(i):
            return right if is_r(i) else left

        def crdev(i):
            return left if is_r(i) else right

        def rs_recv_chunk(i, h):
            return (lax.rem(p + (N - 1 - h), N) if is_r(i)
                    else lax.rem(p + (h + 1), N))

        def own_chunk(i):
            return lax.rem(p + 1, N) if is_r(i) else lax.rem(p + (N - 1), N)

        def ag_send_chunk(i, h):
            return (lax.rem(p + (N + 1 - h), N) if is_r(i)
                    else lax.rem(p + (N - 1 + h), N))

        def pslice(c, i):
            return partial_ref.at[pl.ds(c * CH, CH), pl.ds(i * Q, Q)]

        def oslice(c, i):
            return out_ref.at[pl.ds(c * CH, CH), pl.ds(i * Q, Q)]

        def rs_rdma(i, h):
            return pltpu.make_async_remote_copy(
                src_ref=sendb.at[i], dst_ref=recvb.at[i, h % 2],
                send_sem=rs_ss.at[i, h], recv_sem=rs_rs.at[i, h],
                device_id=(dev(i),), device_id_type=_DevId.MESH)

        def ag_rdma(i, h):
            sl = oslice(ag_send_chunk(i, h), i)
            return pltpu.make_async_remote_copy(
                src_ref=sl, dst_ref=sl,
                send_sem=ag_ss.at[i, h], recv_sem=ag_rs.at[i, h],
                device_id=(dev(i),), device_id_type=_DevId.MESH)

        seeds = {}
        for i in PIPES:
            cp = pltpu.make_async_copy(pslice(p, i), sendb.at[i], ldma.at[i])
            cp.start()
            seeds[i] = cp
        rdmas = {}
        pcs = {}
        for i in PIPES:
            seeds[i].wait()
            r = rs_rdma(i, 0)
            r.start()
            rdmas[i] = r
            cp = pltpu.make_async_copy(
                pslice(rs_recv_chunk(i, 0), i), pbuf.at[i], pdma.at[i])
            cp.start()
            pcs[i] = cp

        for h in range(N - 1):
            for i in PIPES:
                rdmas.pop(i).wait()
                pcs.pop(i).wait()
                sendb[i] = recvb[i, h % 2] + pbuf[i]
                _sem_signal(credit.at[i, h % 2], inc=1,
                            device_id=(crdev(i),),
                            device_id_type=_DevId.MESH)
                if h + 1 < N - 1:
                    if h + 1 >= 2:
                        _sem_wait(credit.at[i, (h + 1) % 2], 1)
                    r = rs_rdma(i, h + 1)
                    r.start()
                    rdmas[i] = r
                    cp = pltpu.make_async_copy(
                        pslice(rs_recv_chunk(i, h + 1), i),
                        pbuf.at[i], pdma.at[i])
                    cp.start()
                    pcs[i] = cp

        sc = s_ref[0, 0]
        stores = {}
        for i in PIPES:
            y = sendb[i].astype(jnp.float32) * sc
            fbuf[i] = y / (1.0 + jnp.exp(-jnp.clip(y, -60.0, 60.0)))
            cp = pltpu.make_async_copy(
                fbuf.at[i], oslice(own_chunk(i), i), ldma.at[i])
            cp.start()
            stores[i] = cp
        for i in PIPES:
            stores[i].wait()
            r = ag_rdma(i, 0)
            r.start()
            rdmas[i] = r

        for h in range(N - 1):
            for i in PIPES:
                rdmas.pop(i).wait()
                if h + 1 < N - 1:
                    r = ag_rdma(i, h + 1)
                    r.start()
                    rdmas[i] = r

        for i in PIPES:
            _sem_wait(credit.at[i, (N - 1) % 2], 1)
            _sem_wait(credit.at[i, N % 2], 1)

    return pl.pallas_call(
        body,
        out_shape=jax.ShapeDtypeStruct((M, Nout), jnp.float32),
        in_specs=[pl.BlockSpec(memory_space=pltpu.SMEM),
                  pl.BlockSpec(memory_space=pl.ANY),
                  pl.BlockSpec(memory_space=pltpu.SMEM)],
        out_specs=pl.BlockSpec(memory_space=pl.ANY),
        scratch_shapes=[
            pltpu.VMEM((4, CH, Q), jnp.int32),
            pltpu.VMEM((4, 2, CH, Q), jnp.int32),
            pltpu.VMEM((4, CH, Q), jnp.int32),
            pltpu.VMEM((4, CH, Q), jnp.float32),
            pltpu.SemaphoreType.DMA((4, N - 1)),
            pltpu.SemaphoreType.DMA((4, N - 1)),
            pltpu.SemaphoreType.DMA((4, N - 1)),
            pltpu.SemaphoreType.DMA((4, N - 1)),
            pltpu.SemaphoreType.REGULAR((4, 2)),
            pltpu.SemaphoreType.DMA((4,)),
            pltpu.SemaphoreType.DMA((4,)),
        ],
    )(meta, partial, s)


# device time: 1508542 ns/iter; 2.0403x vs baseline; 1.0300x over previous
import jax
import jax.numpy as jnp
from jax import lax
from jax.experimental import pallas as pl
from jax.experimental.pallas import tpu as pltpu

N = 32

_sem_signal = getattr(pl, "semaphore_signal", None) or pltpu.semaphore_signal
_sem_wait = getattr(pl, "semaphore_wait", None) or pltpu.semaphore_wait
_DevId = getattr(pl, "DeviceIdType", None) or pltpu.DeviceIdType


def _ring_permutation():
    p_yz = []
    for z in range(4):
        ys = range(4) if z % 2 == 0 else range(3, -1, -1)
        p_yz.extend((y, z) for y in ys)
    cycle = [(0, y, z) for (y, z) in p_yz] + \
            [(1, y, z) for (y, z) in reversed(p_yz)]
    plane = [(0, 0), (1, 0), (1, 1), (0, 1), (0, 2), (1, 2), (1, 3), (0, 3)]
    return [8 * z + plane.index((x, y)) for (x, y, z) in cycle]


_PI = _ring_permutation()
_INV = [0] * N
for _p, _m in enumerate(_PI):
    _INV[_m] = _p


def kernel(x, w_mat, scale_x, scale_w):
    M, K = x.shape
    Nout = w_mat.shape[1]
    CH = M // N
    Q = Nout // 4

    s = (scale_x * scale_w).reshape(1, 1)

    me = lax.axis_index("i")
    p = jnp.asarray(_INV, jnp.int32)[me]
    pi = jnp.asarray(_PI, jnp.int32)
    right_m = pi[lax.rem(p + 1, N)]
    left_m = pi[lax.rem(p + (N - 1), N)]
    meta = jnp.stack([p.astype(jnp.int32), left_m, right_m])

    def body(meta_ref, x_ref, w_ref, s_ref, out_ref,
             sendb, recvb, pbuf, fbuf, xb, wb,
             rs_ss, rs_rs, ag_ss, ag_rs, credit, ldma):
        p = meta_ref[0]
        left = meta_ref[1]
        right = meta_ref[2]

        PIPES = (0, 2, 1, 3)

        def is_r(i):
            return i < 2

        def dev(i):
            return right if is_r(i) else left

        def crdev(i):
            return left if is_r(i) else right

        def rs_recv_chunk(i, h):
            return (lax.rem(p + (N - 1 - h), N) if is_r(i)
                    else lax.rem(p + (h + 1), N))

        def own_chunk(i):
            return lax.rem(p + 1, N) if is_r(i) else lax.rem(p + (N - 1), N)

        def ag_send_chunk(i, h):
            return (lax.rem(p + (N + 1 - h), N) if is_r(i)
                    else lax.rem(p + (N - 1 + h), N))

        def oslice(c, i):
            return out_ref.at[pl.ds(c * CH, CH), pl.ds(i * Q, Q)]

        def pdot(c, i):
            xc = xb[pl.ds(c * CH, CH), :]
            wc = wb[:, i * Q:(i + 1) * Q]
            acc = lax.dot_general(
                xc, wc, dimension_numbers=(((1,), (0,)), ((), ())),
                preferred_element_type=jnp.float32)
            return acc.astype(jnp.int32)

        def rs_rdma(i, h):
            return pltpu.make_async_remote_copy(
                src_ref=sendb.at[i], dst_ref=recvb.at[i, h % 2],
                send_sem=rs_ss.at[i, h], recv_sem=rs_rs.at[i, h],
                device_id=(dev(i),), device_id_type=_DevId.MESH)

        def ag_rdma(i, h):
            sl = oslice(ag_send_chunk(i, h), i)
            return pltpu.make_async_remote_copy(
                src_ref=fbuf.at[i] if h == 0 else sl, dst_ref=sl,
                send_sem=ag_ss.at[i, h], recv_sem=ag_rs.at[i, h],
                device_id=(dev(i),), device_id_type=_DevId.MESH)

        xb[...] = x_ref[...].astype(jnp.bfloat16)
        wb[...] = w_ref[...].astype(jnp.bfloat16)
        rdmas = {}
        for i in PIPES:
            sendb[i] = pdot(p, i)
            r = rs_rdma(i, 0)
            r.start()
            rdmas[i] = r
        for i in PIPES:
            pbuf[i] = pdot(rs_recv_chunk(i, 0), i)

        for h in range(N - 1):
            for i in PIPES:
                rdmas.pop(i).wait()
                sendb[i] = recvb[i, h % 2] + pbuf[i]
                _sem_signal(credit.at[i, h % 2], inc=1,
                            device_id=(crdev(i),),
                            device_id_type=_DevId.MESH)
                if h + 1 < N - 1:
                    if h + 1 >= 2:
                        _sem_wait(credit.at[i, (h + 1) % 2], 1)
                    r = rs_rdma(i, h + 1)
                    r.start()
                    rdmas[i] = r
                    pbuf[i] = pdot(rs_recv_chunk(i, h + 1), i)

        sc = s_ref[0, 0]
        stores = {}
        for i in PIPES:
            y = sendb[i].astype(jnp.float32) * sc
            fbuf[i] = y / (1.0 + jnp.exp(-jnp.clip(y, -60.0, 60.0)))
            r = ag_rdma(i, 0)
            r.start()
            rdmas[i] = r
            cp = pltpu.make_async_copy(
                fbuf.at[i], oslice(own_chunk(i), i), ldma.at[i])
            cp.start()
            stores[i] = cp

        for h in range(N - 1):
            for i in PIPES:
                rdmas.pop(i).wait()
                if h + 1 < N - 1:
                    r = ag_rdma(i, h + 1)
                    r.start()
                    rdmas[i] = r

        for i in PIPES:
            stores[i].wait()
        for i in PIPES:
            _sem_wait(credit.at[i, (N - 1) % 2], 1)
            _sem_wait(credit.at[i, N % 2], 1)

    return pl.pallas_call(
        body,
        out_shape=jax.ShapeDtypeStruct((M, Nout), jnp.float32),
        in_specs=[pl.BlockSpec(memory_space=pltpu.SMEM),
                  pl.BlockSpec(memory_space=pltpu.VMEM),
                  pl.BlockSpec(memory_space=pltpu.VMEM),
                  pl.BlockSpec(memory_space=pltpu.SMEM)],
        out_specs=pl.BlockSpec(memory_space=pl.ANY),
        scratch_shapes=[
            pltpu.VMEM((4, CH, Q), jnp.int32),
            pltpu.VMEM((4, 2, CH, Q), jnp.int32),
            pltpu.VMEM((4, CH, Q), jnp.int32),
            pltpu.VMEM((4, CH, Q), jnp.float32),
            pltpu.VMEM((M, K), jnp.bfloat16),
            pltpu.VMEM((K, Nout), jnp.bfloat16),
            pltpu.SemaphoreType.DMA((4, N - 1)),
            pltpu.SemaphoreType.DMA((4, N - 1)),
            pltpu.SemaphoreType.DMA((4, N - 1)),
            pltpu.SemaphoreType.DMA((4, N - 1)),
            pltpu.SemaphoreType.REGULAR((4, 2)),
            pltpu.SemaphoreType.DMA((4,)),
        ],
    )(meta, x, w_mat, s)


# device time: 1501187 ns/iter; 2.0503x vs baseline; 1.0049x over previous
import jax
import jax.numpy as jnp
from jax import lax
from jax.experimental import pallas as pl
from jax.experimental.pallas import tpu as pltpu

N = 32

_sem_signal = getattr(pl, "semaphore_signal", None) or pltpu.semaphore_signal
_sem_wait = getattr(pl, "semaphore_wait", None) or pltpu.semaphore_wait
_DevId = getattr(pl, "DeviceIdType", None) or pltpu.DeviceIdType


def _ring_permutation():
    p_yz = []
    for z in range(4):
        ys = range(4) if z % 2 == 0 else range(3, -1, -1)
        p_yz.extend((y, z) for y in ys)
    cycle = [(0, y, z) for (y, z) in p_yz] + \
            [(1, y, z) for (y, z) in reversed(p_yz)]
    plane = [(0, 0), (1, 0), (1, 1), (0, 1), (0, 2), (1, 2), (1, 3), (0, 3)]
    return [8 * z + plane.index((x, y)) for (x, y, z) in cycle]


_PI = _ring_permutation()
_INV = [0] * N
for _p, _m in enumerate(_PI):
    _INV[_m] = _p


def kernel(x, w_mat, scale_x, scale_w):
    M, K = x.shape
    Nout = w_mat.shape[1]
    CH = M // N
    Q = Nout // 4

    s = (scale_x * scale_w).reshape(1, 1)

    me = lax.axis_index("i")
    p = jnp.asarray(_INV, jnp.int32)[me]
    pi = jnp.asarray(_PI, jnp.int32)
    right_m = pi[lax.rem(p + 1, N)]
    left_m = pi[lax.rem(p + (N - 1), N)]
    meta = jnp.stack([p.astype(jnp.int32), left_m, right_m])

    def body(meta_ref, x_ref, w_ref, s_ref, out_ref,
             sendb, recvb, pbuf, fbuf, xb, wb,
             rs_ss, rs_rs, ag_ss, ag_rs, credit, ldma):
        p = meta_ref[0]
        left = meta_ref[1]
        right = meta_ref[2]

        PIPES = (0, 2, 1, 3)

        def is_r(i):
            return i < 2

        def dev(i):
            return right if is_r(i) else left

        def crdev(i):
            return left if is_r(i) else right

        def rs_recv_chunk(i, h):
            return (lax.rem(p + (N - 1 - h), N) if is_r(i)
                    else lax.rem(p + (h + 1), N))

        def own_chunk(i):
            return lax.rem(p + 1, N) if is_r(i) else lax.rem(p + (N - 1), N)

        def ag_send_chunk(i, h):
            return (lax.rem(p + (N + 1 - h), N) if is_r(i)
                    else lax.rem(p + (N - 1 + h), N))

        def oslice(c, i):
            return out_ref.at[pl.ds(c * CH, CH), pl.ds(i * Q, Q)]

        def pdot(c, i):
            xc = xb[pl.ds(c * CH, CH), :]
            wc = wb[:, i * Q:(i + 1) * Q]
            acc = lax.dot_general(
                xc, wc, dimension_numbers=(((1,), (0,)), ((), ())),
                preferred_element_type=jnp.float32)
            return acc.astype(jnp.int32)

        def rs_rdma(i, h):
            return pltpu.make_async_remote_copy(
                src_ref=sendb.at[i], dst_ref=recvb.at[i, h % 2],
                send_sem=rs_ss.at[i, h], recv_sem=rs_rs.at[i, h],
                device_id=(dev(i),), device_id_type=_DevId.MESH)

        def ag_rdma(i, h):
            sl = oslice(ag_send_chunk(i, h), i)
            return pltpu.make_async_remote_copy(
                src_ref=fbuf.at[i] if h == 0 else sl, dst_ref=sl,
                send_sem=ag_ss.at[i, h], recv_sem=ag_rs.at[i, h],
                device_id=(dev(i),), device_id_type=_DevId.MESH)

        bsem = pltpu.get_barrier_semaphore()
        _sem_signal(bsem, inc=1, device_id=(left,),
                    device_id_type=_DevId.MESH)
        _sem_signal(bsem, inc=1, device_id=(right,),
                    device_id_type=_DevId.MESH)
        _sem_wait(bsem, 2)

        xb[...] = x_ref[...].astype(jnp.bfloat16)
        wb[...] = w_ref[...].astype(jnp.bfloat16)
        rdmas = {}
        for i in PIPES:
            sendb[i] = pdot(p, i)
            r = rs_rdma(i, 0)
            r.start()
            rdmas[i] = r
        for i in PIPES:
            pbuf[i] = pdot(rs_recv_chunk(i, 0), i)

        for h in range(N - 1):
            for i in PIPES:
                rdmas.pop(i).wait()
                sendb[i] = recvb[i, h % 2] + pbuf[i]
                _sem_signal(credit.at[i, h % 2], inc=1,
                            device_id=(crdev(i),),
                            device_id_type=_DevId.MESH)
                if h + 1 < N - 1:
                    if h + 1 >= 2:
                        _sem_wait(credit.at[i, (h + 1) % 2], 1)
                    r = rs_rdma(i, h + 1)
                    r.start()
                    rdmas[i] = r
                    pbuf[i] = pdot(rs_recv_chunk(i, h + 1), i)

        sc = s_ref[0, 0]
        stores = {}
        for i in PIPES:
            y = sendb[i].astype(jnp.float32) * sc
            fbuf[i] = y / (1.0 + jnp.exp(-jnp.clip(y, -60.0, 60.0)))
            r = ag_rdma(i, 0)
            r.start()
            rdmas[i] = r
            cp = pltpu.make_async_copy(
                fbuf.at[i], oslice(own_chunk(i), i), ldma.at[i])
            cp.start()
            stores[i] = cp

        for h in range(N - 1):
            for i in PIPES:
                rdmas.pop(i).wait()
                if h + 1 < N - 1:
                    r = ag_rdma(i, h + 1)
                    r.start()
                    rdmas[i] = r

        for i in PIPES:
            stores[i].wait()
        for i in PIPES:
            _sem_wait(credit.at[i, (N - 1) % 2], 1)
            _sem_wait(credit.at[i, N % 2], 1)

    return pl.pallas_call(
        body,
        out_shape=jax.ShapeDtypeStruct((M, Nout), jnp.float32),
        in_specs=[pl.BlockSpec(memory_space=pltpu.SMEM),
                  pl.BlockSpec(memory_space=pltpu.VMEM),
                  pl.BlockSpec(memory_space=pltpu.VMEM),
                  pl.BlockSpec(memory_space=pltpu.SMEM)],
        out_specs=pl.BlockSpec(memory_space=pl.ANY),
        scratch_shapes=[
            pltpu.VMEM((4, CH, Q), jnp.int32),
            pltpu.VMEM((4, 2, CH, Q), jnp.int32),
            pltpu.VMEM((4, CH, Q), jnp.int32),
            pltpu.VMEM((4, CH, Q), jnp.float32),
            pltpu.VMEM((M, K), jnp.bfloat16),
            pltpu.VMEM((K, Nout), jnp.bfloat16),
            pltpu.SemaphoreType.DMA((4, N - 1)),
            pltpu.SemaphoreType.DMA((4, N - 1)),
            pltpu.SemaphoreType.DMA((4, N - 1)),
            pltpu.SemaphoreType.DMA((4, N - 1)),
            pltpu.SemaphoreType.REGULAR((4, 2)),
            pltpu.SemaphoreType.DMA((4,)),
        ],
        compiler_params=pltpu.CompilerParams(collective_id=0),
    )(meta, x, w_mat, s)
